# baseline (speedup 1.0000x reference)
import functools

import jax
import jax.numpy as jnp
from jax.experimental import pallas as pl
from jax.experimental.pallas import tpu as pltpu

N = 10000
E = 320000
D_LAT_E = 8
D_LAT_N = 64


def _fc_body(g_ref, w_ref, b_ref, o_ref):
    acc = jnp.dot(g_ref[...], w_ref[...], preferred_element_type=jnp.float32)
    o_ref[...] = jax.nn.relu(acc + b_ref[...])


def _fc(graph, w, b):
    n = graph.shape[0]
    blk = 400
    return pl.pallas_call(
        _fc_body,
        grid=(n // blk,),
        in_specs=[
            pl.BlockSpec((blk, graph.shape[1]), lambda i: (i, 0)),
            pl.BlockSpec((w.shape[0], w.shape[1]), lambda i: (0, 0)),
            pl.BlockSpec((w.shape[1],), lambda i: (0,)),
        ],
        out_specs=pl.BlockSpec((blk, w.shape[1]), lambda i: (i, 0)),
        out_shape=jax.ShapeDtypeStruct((n, w.shape[1]), jnp.float32),
    )(graph, w, b)


def kernel(x, edge_index, edge_attr, W_conv, b_conv, W_edge, b_edge, W_node, b_node):
    edge_props = jax.nn.relu(edge_attr @ W_edge + b_edge)
    n = x.shape[0]
    loop = jnp.arange(n, dtype=edge_index.dtype)
    row = jnp.concatenate([edge_index[0], loop], axis=0)
    col = jnp.concatenate([edge_index[1], loop], axis=0)
    xw = x @ W_conv
    collection = []
    for i in range(D_LAT_E):
        ew = jnp.concatenate([edge_props[:, i], jnp.ones((n,), dtype=x.dtype)], axis=0)
        deg = jnp.zeros((n,), dtype=x.dtype).at[col].add(ew)
        dis = jnp.where(deg > 0, jax.lax.rsqrt(deg), 0.0)
        norm = dis[row] * ew * dis[col]
        msgs = xw[row] * norm[:, None]
        out = jnp.zeros((n, xw.shape[1]), dtype=x.dtype).at[col].add(msgs)
        collection.append(out + b_conv)
    graph_step_0 = jnp.concatenate(collection, axis=1)
    return _fc(graph_step_0, W_node, b_node)


# speedup vs baseline: 29.6841x; 29.6841x over previous
import functools

import jax
import jax.numpy as jnp
from jax import lax
from jax.experimental import pallas as pl
from jax.experimental.pallas import tpu as pltpu
from jax.experimental.pallas import tpu_sc as plsc

N = 10000
E = 320000
D_IN = 128
D_LAT_N = 64
D_LAT_E = 8
D_EDGE = 16
D_OUT = 128

NTILE = 16
NCORE = 2
NPAD = 10240
CHUNK = 80
BLK_D = 400
BLK_M = 160
ROWS_PT = NPAD // NTILE
EDGES_PT = E // NTILE
EDGES_PT_DEG = E // (NTILE * NCORE)


def _edge_props_body(ea_ref, w_ref, b_ref, o_ref):
    g = jax.nn.relu(
        jnp.dot(ea_ref[...], w_ref[...], preferred_element_type=jnp.float32)
        + b_ref[...]
    )
    o_ref[...] = jnp.concatenate(
        [g, jnp.zeros((g.shape[0], D_LAT_E), jnp.float32)], axis=1)


def _edge_props(edge_attr, W_edge, b_edge):
    blk = 8000
    return pl.pallas_call(
        _edge_props_body,
        grid=(E // blk,),
        in_specs=[
            pl.BlockSpec((blk, D_EDGE), lambda i: (i, 0)),
            pl.BlockSpec((D_EDGE, D_LAT_E), lambda i: (0, 0)),
            pl.BlockSpec((D_LAT_E,), lambda i: (0,)),
        ],
        out_specs=pl.BlockSpec((blk, 2 * D_LAT_E), lambda i: (i, 0)),
        out_shape=jax.ShapeDtypeStruct((E, 2 * D_LAT_E), jnp.float32),
    )(edge_attr, W_edge, b_edge)


def _xw_body(x_ref, w_ref, o_ref):
    o_ref[...] = jnp.dot(x_ref[...], w_ref[...], preferred_element_type=jnp.float32)


def _xw(x, W_conv):
    blk = 2000
    return pl.pallas_call(
        _xw_body,
        grid=(N // blk,),
        in_specs=[
            pl.BlockSpec((blk, D_IN), lambda i: (i, 0)),
            pl.BlockSpec((D_IN, D_LAT_N), lambda i: (0, 0)),
        ],
        out_specs=pl.BlockSpec((blk, D_LAT_N), lambda i: (i, 0)),
        out_shape=jax.ShapeDtypeStruct((N, D_LAT_N), jnp.float32),
    )(x, W_conv)


def _post_deg_body(dp_ref, xw_ref, dis_ref, xws_ref):
    deg = 1.0 + dp_ref[0][:, :D_LAT_E] + dp_ref[1][:, :D_LAT_E]
    dis = lax.rsqrt(deg)
    dis_ref[...] = jnp.concatenate([dis, dis], axis=1)
    xw = xw_ref[...]
    for q in range(4):
        for c in range(2):
            xws_ref[q, :, c * D_LAT_N:(c + 1) * D_LAT_N] = (
                dis[:, 2 * q + c:2 * q + c + 1] * xw)


def _post_deg(degpart, xw):
    blk = 400
    return pl.pallas_call(
        _post_deg_body,
        grid=(N // blk,),
        in_specs=[
            pl.BlockSpec((NCORE, blk, 2 * D_LAT_N), lambda i: (0, i, 0)),
            pl.BlockSpec((blk, D_LAT_N), lambda i: (i, 0)),
        ],
        out_specs=[
            pl.BlockSpec((blk, 2 * D_LAT_E), lambda i: (i, 0)),
            pl.BlockSpec((4, blk, 2 * D_LAT_N), lambda i: (0, i, 0)),
        ],
        out_shape=[
            jax.ShapeDtypeStruct((N, 2 * D_LAT_E), jnp.float32),
            jax.ShapeDtypeStruct((4, N, 2 * D_LAT_N), jnp.float32),
        ],
    )(degpart, xw)


def _final_body(sed_ref, dis_ref, xw_ref, wn_ref, bc_ref, bn_ref, o_ref):
    xw = xw_ref[...]
    bc = bc_ref[...][None, :]
    acc = jnp.broadcast_to(bn_ref[...][None, :], o_ref.shape).astype(jnp.float32)
    for q in range(4):
        sq = sed_ref[q]
        for cc in range(2):
            ch = 2 * q + cc
            d = dis_ref[:, ch][:, None]
            g = d * sq[:, cc * 64:(cc + 1) * 64] + (d * d) * xw + bc
            acc = acc + jnp.dot(
                g, wn_ref[ch * 64:(ch + 1) * 64, :],
                preferred_element_type=jnp.float32,
            )
    o_ref[...] = jax.nn.relu(acc)


def _final(sedge, dis, xw, W_node, b_conv, b_node):
    blk = 400
    return pl.pallas_call(
        _final_body,
        grid=(N // blk,),
        in_specs=[
            pl.BlockSpec((4, blk, 2 * D_LAT_N), lambda i: (0, i, 0)),
            pl.BlockSpec((blk, 2 * D_LAT_E), lambda i: (i, 0)),
            pl.BlockSpec((blk, D_LAT_N), lambda i: (i, 0)),
            pl.BlockSpec((D_LAT_E * D_LAT_N, D_OUT), lambda i: (0, 0)),
            pl.BlockSpec((D_LAT_N,), lambda i: (0,)),
            pl.BlockSpec((D_OUT,), lambda i: (0,)),
        ],
        out_specs=pl.BlockSpec((blk, D_OUT), lambda i: (i, 0)),
        out_shape=jax.ShapeDtypeStruct((N, D_OUT), jnp.float32),
    )(sedge, dis, xw, W_node, b_conv, b_node)


def _repack_idx(flat_ref, idx2_ref, nchunk):
    for s in range(nchunk):
        for j in range(CHUNK // 16):
            idx2_ref[s, pl.ds(j * 16, 16)] = flat_ref[pl.ds(s * CHUNK + j * 16, 16)]


def _deg_sc(col, ew16, zeros128):
    mesh = plsc.VectorSubcoreMesh(core_axis_name="c", subcore_axis_name="s")

    @functools.partial(
        pl.kernel,
        out_type=jax.ShapeDtypeStruct((NCORE, NPAD, 2 * D_LAT_N), jnp.float32),
        mesh=mesh,
        scratch_types=[
            pltpu.MemorySpace.VMEM_SHARED((NPAD, 2 * D_LAT_N), jnp.float32),
            pltpu.VMEM((BLK_D,), jnp.int32),
            pltpu.VMEM((BLK_D // CHUNK, CHUNK), jnp.int32),
            pltpu.VMEM((BLK_D * 2 * D_LAT_E,), jnp.float32),
            pltpu.VMEM((CHUNK, 2 * D_LAT_N), jnp.float32),
            pltpu.SemaphoreType.DMA,
            pltpu.SemaphoreType.DMA,
        ],
    )
    def k(col_hbm, ew_hbm, zeros_hbm, degpart_hbm, deg_sh, colb1, colb, ewb,
          ewpad, lsem, ssem):
        core = lax.axis_index("c")
        tile = lax.axis_index("s")
        rbase = tile * ROWS_PT
        pltpu.sync_copy(zeros_hbm, deg_sh.at[pl.ds(rbase, ROWS_PT), :])

        def zpad(i, _):
            for j in range(8):
                ewpad[i, pl.ds(j * 16, 16)] = jnp.zeros((16,), jnp.float32)
            return 0
        lax.fori_loop(0, CHUNK, zpad, 0)
        plsc.subcore_barrier()

        nblocks = EDGES_PT_DEG // BLK_D
        ebase0 = core * (E // NCORE) + tile * EDGES_PT_DEG

        def body(b, _):
            d1 = pltpu.async_copy(
                col_hbm.at[pl.ds(ebase0 + b * BLK_D, BLK_D)], colb1, lsem)
            d2 = pltpu.async_copy(
                ew_hbm.at[pl.ds((ebase0 + b * BLK_D) * 16, BLK_D * 16)],
                ewb, lsem)
            d1.wait()
            d2.wait()
            _repack_idx(colb1, colb, BLK_D // CHUNK)

            for s in range(BLK_D // CHUNK):
                def pbody(i, _, s=s):
                    ewpad[i, pl.ds(0, 16)] = ewb[pl.ds((s * CHUNK + i) * 16, 16)]
                    return 0
                lax.fori_loop(0, CHUNK, pbody, 0)
                pltpu.async_copy(
                    ewpad, deg_sh.at[colb.at[s]], ssem, add=True).wait()
            return 0

        lax.fori_loop(0, nblocks, body, 0)
        plsc.subcore_barrier()
        pltpu.sync_copy(
            deg_sh.at[pl.ds(rbase, ROWS_PT), :],
            degpart_hbm.at[core, pl.ds(rbase, ROWS_PT), :])

    return k(col, ew16, zeros128)


def _msg_sc(row, col, ew16, xws4, zeros128):
    mesh = plsc.VectorSubcoreMesh(core_axis_name="c", subcore_axis_name="s")

    @functools.partial(
        pl.kernel,
        out_type=jax.ShapeDtypeStruct((4, NPAD, 2 * D_LAT_N), jnp.float32),
        mesh=mesh,
        scratch_types=[
            pltpu.MemorySpace.VMEM_SHARED((NPAD, 2 * D_LAT_N), jnp.float32),
            pltpu.VMEM((BLK_M,), jnp.int32),
            pltpu.VMEM((BLK_M,), jnp.int32),
            pltpu.VMEM((BLK_M // CHUNK, CHUNK), jnp.int32),
            pltpu.VMEM((BLK_M // CHUNK, CHUNK), jnp.int32),
            pltpu.VMEM((BLK_M * 2 * D_LAT_E,), jnp.float32),
            pltpu.VMEM((BLK_M, 2 * D_LAT_N), jnp.float32),
            pltpu.VMEM((BLK_M, 2 * D_LAT_N), jnp.float32),
            pltpu.SemaphoreType.DMA,
            pltpu.SemaphoreType.DMA,
            pltpu.SemaphoreType.DMA,
        ],
    )
    def k(row_hbm, col_hbm, ew_hbm, xws_hbm, zeros_hbm, sedge_hbm,
          out_sh, rowb1, colb1, rowb, colb, ewb, xwsb, msgb,
          lsem, gsem, ssem):
        core = lax.axis_index("c")
        tile = lax.axis_index("s")
        rbase = tile * ROWS_PT
        ebase0 = tile * EDGES_PT
        nblocks = EDGES_PT // BLK_M

        def run_pass(q):
            pltpu.sync_copy(zeros_hbm, out_sh.at[pl.ds(rbase, ROWS_PT), :])
            plsc.subcore_barrier()

            def body(b, _):
                est = ebase0 + b * BLK_M
                d1 = pltpu.async_copy(row_hbm.at[pl.ds(est, BLK_M)], rowb1, lsem)
                d2 = pltpu.async_copy(col_hbm.at[pl.ds(est, BLK_M)], colb1, lsem)
                d3 = pltpu.async_copy(
                    ew_hbm.at[pl.ds(est * 16, BLK_M * 16)], ewb, lsem)
                d1.wait()
                d2.wait()
                d3.wait()
                _repack_idx(rowb1, rowb, BLK_M // CHUNK)
                _repack_idx(colb1, colb, BLK_M // CHUNK)
                descs = []
                for s in range(BLK_M // CHUNK):
                    descs.append(pltpu.async_copy(
                        xws_hbm.at[q].at[rowb.at[s]],
                        xwsb.at[pl.ds(s * CHUNK, CHUNK), :], gsem))
                for d in descs:
                    d.wait()

                def mbody(g, _):
                    for i in range(16):
                        e = g * 16 + i
                        wv = ewb[pl.ds(e * 16, 16)]
                        a0 = wv[2 * q]
                        a1 = wv[2 * q + 1]
                        for r in range(4):
                            msgb[e, pl.ds(r * 16, 16)] = (
                                a0 * xwsb[e, pl.ds(r * 16, 16)])
                            msgb[e, pl.ds(64 + r * 16, 16)] = (
                                a1 * xwsb[e, pl.ds(64 + r * 16, 16)])
                    return 0
                lax.fori_loop(0, BLK_M // 16, mbody, 0)

                sdescs = []
                for s in range(BLK_M // CHUNK):
                    sdescs.append(pltpu.async_copy(
                        msgb.at[pl.ds(s * CHUNK, CHUNK), :],
                        out_sh.at[colb.at[s]], ssem, add=True))
                for d in sdescs:
                    d.wait()
                return 0

            lax.fori_loop(0, nblocks, body, 0)
            plsc.subcore_barrier()
            pltpu.sync_copy(
                out_sh.at[pl.ds(rbase, ROWS_PT), :],
                sedge_hbm.at[q, pl.ds(rbase, ROWS_PT), :])
            plsc.subcore_barrier()

        for cv in range(NCORE):
            @pl.when(core == cv)
            def _():
                for p in range(2):
                    run_pass(2 * cv + p)

    return k(row, col, ew16, xws4, zeros128)


def kernel(x, edge_index, edge_attr, W_conv, b_conv, W_edge, b_edge, W_node, b_node):
    row = edge_index[0]
    col = edge_index[1]
    ew16 = _edge_props(edge_attr, W_edge, b_edge)
    xw = _xw(x, W_conv)
    zeros128 = jnp.zeros((ROWS_PT, 2 * D_LAT_N), jnp.float32)
    ewf = ew16.reshape(E * 2 * D_LAT_E)
    degpart = _deg_sc(col, ewf, zeros128)
    dis16, xws4 = _post_deg(degpart, xw)
    sedge = _msg_sc(row, col, ewf, xws4, zeros128)
    return _final(sedge, dis16, xw, W_node, b_conv, b_node)


# speedup vs baseline: 39.5663x; 1.3329x over previous
import functools

import jax
import jax.numpy as jnp
from jax import lax
from jax.experimental import pallas as pl
from jax.experimental.pallas import tpu as pltpu
from jax.experimental.pallas import tpu_sc as plsc

N = 10000
E = 320000
D_IN = 128
D_LAT_N = 64
D_LAT_E = 8
D_EDGE = 16
D_OUT = 128

NTILE = 16
NCORE = 2
NPAD = 10240
CHUNK = 80
BLK_D = 400
BLK_M = 80
ROWS_PT = NPAD // NTILE
EDGES_PT = E // NTILE
EDGES_PT_DEG = E // (NTILE * NCORE)


def _edge_props_body(ea_ref, w_ref, b_ref, o_ref):
    g = jax.nn.relu(
        jnp.dot(ea_ref[...], w_ref[...], preferred_element_type=jnp.float32)
        + b_ref[...]
    )
    o_ref[...] = jnp.concatenate(
        [g, jnp.zeros((g.shape[0], D_LAT_E), jnp.float32)], axis=1)


def _edge_props(edge_attr, W_edge, b_edge):
    blk = 8000
    return pl.pallas_call(
        _edge_props_body,
        grid=(E // blk,),
        in_specs=[
            pl.BlockSpec((blk, D_EDGE), lambda i: (i, 0)),
            pl.BlockSpec((D_EDGE, D_LAT_E), lambda i: (0, 0)),
            pl.BlockSpec((D_LAT_E,), lambda i: (0,)),
        ],
        out_specs=pl.BlockSpec((blk, 2 * D_LAT_E), lambda i: (i, 0)),
        out_shape=jax.ShapeDtypeStruct((E, 2 * D_LAT_E), jnp.float32),
    )(edge_attr, W_edge, b_edge)


def _xw_body(x_ref, w_ref, o_ref):
    o_ref[...] = jnp.dot(x_ref[...], w_ref[...], preferred_element_type=jnp.float32)


def _xw(x, W_conv):
    blk = 2000
    return pl.pallas_call(
        _xw_body,
        grid=(N // blk,),
        in_specs=[
            pl.BlockSpec((blk, D_IN), lambda i: (i, 0)),
            pl.BlockSpec((D_IN, D_LAT_N), lambda i: (0, 0)),
        ],
        out_specs=pl.BlockSpec((blk, D_LAT_N), lambda i: (i, 0)),
        out_shape=jax.ShapeDtypeStruct((N, D_LAT_N), jnp.float32),
    )(x, W_conv)


def _post_deg_body(dp_ref, xw_ref, dis_ref, xws_ref):
    deg = 1.0 + dp_ref[0][:, :D_LAT_E] + dp_ref[1][:, :D_LAT_E]
    dis = lax.rsqrt(deg)
    dis_ref[...] = jnp.concatenate([dis, dis], axis=1)
    xw = xw_ref[...]
    for q in range(4):
        for c in range(2):
            xws_ref[q, :, c * D_LAT_N:(c + 1) * D_LAT_N] = (
                dis[:, 2 * q + c:2 * q + c + 1] * xw)


def _post_deg(degpart, xw):
    blk = 400
    return pl.pallas_call(
        _post_deg_body,
        grid=(N // blk,),
        in_specs=[
            pl.BlockSpec((NCORE, blk, 2 * D_LAT_N), lambda i: (0, i, 0)),
            pl.BlockSpec((blk, D_LAT_N), lambda i: (i, 0)),
        ],
        out_specs=[
            pl.BlockSpec((blk, 2 * D_LAT_E), lambda i: (i, 0)),
            pl.BlockSpec((4, blk, 2 * D_LAT_N), lambda i: (0, i, 0)),
        ],
        out_shape=[
            jax.ShapeDtypeStruct((N, 2 * D_LAT_E), jnp.float32),
            jax.ShapeDtypeStruct((4, N, 2 * D_LAT_N), jnp.float32),
        ],
    )(degpart, xw)


def _final_body(sed_ref, dis_ref, xw_ref, wn_ref, bc_ref, bn_ref, o_ref):
    xw = xw_ref[...]
    bc = bc_ref[...][None, :]
    acc = jnp.broadcast_to(bn_ref[...][None, :], o_ref.shape).astype(jnp.float32)
    for q in range(4):
        sq = sed_ref[q]
        for cc in range(2):
            ch = 2 * q + cc
            d = dis_ref[:, ch][:, None]
            g = d * sq[:, cc * 64:(cc + 1) * 64] + (d * d) * xw + bc
            acc = acc + jnp.dot(
                g, wn_ref[ch * 64:(ch + 1) * 64, :],
                preferred_element_type=jnp.float32,
            )
    o_ref[...] = jax.nn.relu(acc)


def _final(sedge, dis, xw, W_node, b_conv, b_node):
    blk = 400
    return pl.pallas_call(
        _final_body,
        grid=(N // blk,),
        in_specs=[
            pl.BlockSpec((4, blk, 2 * D_LAT_N), lambda i: (0, i, 0)),
            pl.BlockSpec((blk, 2 * D_LAT_E), lambda i: (i, 0)),
            pl.BlockSpec((blk, D_LAT_N), lambda i: (i, 0)),
            pl.BlockSpec((D_LAT_E * D_LAT_N, D_OUT), lambda i: (0, 0)),
            pl.BlockSpec((D_LAT_N,), lambda i: (0,)),
            pl.BlockSpec((D_OUT,), lambda i: (0,)),
        ],
        out_specs=pl.BlockSpec((blk, D_OUT), lambda i: (i, 0)),
        out_shape=jax.ShapeDtypeStruct((N, D_OUT), jnp.float32),
    )(sedge, dis, xw, W_node, b_conv, b_node)


def _repack_idx(flat_ref, idx2_ref, nchunk):
    for s in range(nchunk):
        for j in range(CHUNK // 16):
            idx2_ref[s, pl.ds(j * 16, 16)] = flat_ref[pl.ds(s * CHUNK + j * 16, 16)]


def _deg_sc(col, ew16, zeros128):
    mesh = plsc.VectorSubcoreMesh(core_axis_name="c", subcore_axis_name="s")

    @functools.partial(
        pl.kernel,
        out_type=jax.ShapeDtypeStruct((NCORE, NPAD, 2 * D_LAT_N), jnp.float32),
        mesh=mesh,
        scratch_types=[
            pltpu.MemorySpace.VMEM_SHARED((NPAD, 2 * D_LAT_N), jnp.float32),
            pltpu.VMEM((BLK_D,), jnp.int32),
            pltpu.VMEM((BLK_D // CHUNK, CHUNK), jnp.int32),
            pltpu.VMEM((BLK_D * 2 * D_LAT_E,), jnp.float32),
            pltpu.VMEM((CHUNK, 2 * D_LAT_N), jnp.float32),
            pltpu.SemaphoreType.DMA,
            pltpu.SemaphoreType.DMA,
        ],
    )
    def k(col_hbm, ew_hbm, zeros_hbm, degpart_hbm, deg_sh, colb1, colb, ewb,
          ewpad, lsem, ssem):
        core = lax.axis_index("c")
        tile = lax.axis_index("s")
        rbase = tile * ROWS_PT
        pltpu.sync_copy(zeros_hbm, deg_sh.at[pl.ds(rbase, ROWS_PT), :])

        def zpad(i, _):
            for j in range(8):
                ewpad[i, pl.ds(j * 16, 16)] = jnp.zeros((16,), jnp.float32)
            return 0
        lax.fori_loop(0, CHUNK, zpad, 0)
        plsc.subcore_barrier()

        nblocks = EDGES_PT_DEG // BLK_D
        ebase0 = core * (E // NCORE) + tile * EDGES_PT_DEG

        def body(b, _):
            d1 = pltpu.async_copy(
                col_hbm.at[pl.ds(ebase0 + b * BLK_D, BLK_D)], colb1, lsem)
            d2 = pltpu.async_copy(
                ew_hbm.at[pl.ds((ebase0 + b * BLK_D) * 16, BLK_D * 16)],
                ewb, lsem)
            d1.wait()
            d2.wait()
            _repack_idx(colb1, colb, BLK_D // CHUNK)

            for s in range(BLK_D // CHUNK):
                def pbody(i, _, s=s):
                    ewpad[i, pl.ds(0, 16)] = ewb[pl.ds((s * CHUNK + i) * 16, 16)]
                    return 0
                lax.fori_loop(0, CHUNK, pbody, 0)
                pltpu.async_copy(
                    ewpad, deg_sh.at[colb.at[s]], ssem, add=True).wait()
            return 0

        lax.fori_loop(0, nblocks, body, 0)
        plsc.subcore_barrier()
        pltpu.sync_copy(
            deg_sh.at[pl.ds(rbase, ROWS_PT), :],
            degpart_hbm.at[core, pl.ds(rbase, ROWS_PT), :])

    return k(col, ew16, zeros128)


def _msg_sc(row, col, ewf, xws4, zeros128):
    mesh = plsc.VectorSubcoreMesh(core_axis_name="c", subcore_axis_name="s")
    nblocks = EDGES_PT // BLK_M

    @functools.partial(
        pl.kernel,
        out_type=jax.ShapeDtypeStruct((4, NPAD, 2 * D_LAT_N), jnp.float32),
        mesh=mesh,
        scratch_types=[
            pltpu.MemorySpace.VMEM_SHARED((NPAD, 2 * D_LAT_N), jnp.float32),
            pltpu.VMEM((2, 1, BLK_M), jnp.int32),
            pltpu.VMEM((4, 1, BLK_M), jnp.int32),
            pltpu.VMEM((BLK_M * 2 * D_LAT_E,), jnp.float32),
            pltpu.VMEM((BLK_M * 2 * D_LAT_E,), jnp.float32),
            pltpu.VMEM((BLK_M, 2 * D_LAT_N), jnp.float32),
            pltpu.VMEM((BLK_M, 2 * D_LAT_N), jnp.float32),
            pltpu.VMEM((BLK_M, 2 * D_LAT_N), jnp.float32),
            pltpu.VMEM((BLK_M, 2 * D_LAT_N), jnp.float32),
            pltpu.SemaphoreType.DMA,
            pltpu.SemaphoreType.DMA,
            pltpu.SemaphoreType.DMA,
            pltpu.SemaphoreType.DMA,
            pltpu.SemaphoreType.DMA,
            pltpu.SemaphoreType.DMA,
        ],
    )
    def k(row_hbm, col_hbm, ew_hbm, xws_hbm, zeros_hbm, sedge_hbm,
          out_sh, rowb, colb, ewb0, ewb1, xwsb0, xwsb1, msgb0, msgb1,
          lsem0, lsem1, gsem0, gsem1, ssem0, ssem1):
        core = lax.axis_index("c")
        tile = lax.axis_index("s")
        rbase = tile * ROWS_PT
        ebase0 = tile * EDGES_PT
        ewb = (ewb0, ewb1)
        xwsb = (xwsb0, xwsb1)
        msgb = (msgb0, msgb1)
        lsem = (lsem0, lsem1)
        gsem = (gsem0, gsem1)
        ssem = (ssem0, ssem1)
        zrows = zeros_hbm.at[pl.ds(0, BLK_M), :]

        def issue_l(b, k_):
            est = ebase0 + b * BLK_M
            pltpu.async_copy(row_hbm.at[pl.ds(est, BLK_M)],
                             rowb.at[k_, 0], lsem[k_])
            pltpu.async_copy(col_hbm.at[pl.ds(est, BLK_M)],
                             colb.at[lax.rem(b, 4), 0], lsem[k_])
            pltpu.async_copy(
                ew_hbm.at[pl.ds(est * 2 * D_LAT_E, BLK_M * 2 * D_LAT_E)],
                ewb[k_], lsem[k_])

        def drain_l(k_):
            pltpu.make_async_copy(row_hbm.at[pl.ds(0, BLK_M)],
                                  rowb.at[k_, 0], lsem[k_]).wait()
            pltpu.make_async_copy(col_hbm.at[pl.ds(0, BLK_M)],
                                  colb.at[0, 0], lsem[k_]).wait()
            pltpu.make_async_copy(
                ew_hbm.at[pl.ds(0, BLK_M * 2 * D_LAT_E)],
                ewb[k_], lsem[k_]).wait()

        def run_pass(q):
            def issue_g(k_):
                pltpu.async_copy(xws_hbm.at[q].at[rowb.at[k_, 0]],
                                 xwsb[k_], gsem[k_])

            def drain_g(k_):
                pltpu.make_async_copy(zrows, xwsb[k_], gsem[k_]).wait()

            def issue_s(b, k_):
                pltpu.async_copy(msgb[k_],
                                 out_sh.at[colb.at[lax.rem(b, 4), 0]],
                                 ssem[k_], add=True)

            def drain_s(k_):
                pltpu.make_async_copy(zrows, msgb[k_], ssem[k_]).wait()

            def compute(k_):
                def mbody(g, _):
                    for i in range(16):
                        e = g * 16 + i
                        wv = ewb[k_][pl.ds(e * 2 * D_LAT_E, 16)]
                        a0 = wv[2 * q]
                        a1 = wv[2 * q + 1]
                        for r in range(4):
                            msgb[k_][e, pl.ds(r * 16, 16)] = (
                                a0 * xwsb[k_][e, pl.ds(r * 16, 16)])
                            msgb[k_][e, pl.ds(64 + r * 16, 16)] = (
                                a1 * xwsb[k_][e, pl.ds(64 + r * 16, 16)])
                    return 0
                lax.fori_loop(0, BLK_M // 16, mbody, 0)

            pltpu.sync_copy(zeros_hbm, out_sh.at[pl.ds(rbase, ROWS_PT), :])
            plsc.subcore_barrier()

            issue_l(0, 0)
            drain_l(0)
            issue_g(0)
            issue_l(1, 1)

            def body(bb, _):
                for k_ in range(2):
                    b = 2 * bb + k_
                    kn = 1 - k_
                    drain_g(k_)

                    @pl.when(b + 1 < nblocks)
                    def _():
                        drain_l(kn)
                        issue_g(kn)

                    @pl.when(b >= 2)
                    def _():
                        drain_s(k_)

                    compute(k_)
                    issue_s(b, k_)

                    @pl.when(b + 2 < nblocks)
                    def _():
                        issue_l(b + 2, k_)
                return 0

            lax.fori_loop(0, nblocks // 2, body, 0)
            drain_s(0)
            drain_s(1)
            plsc.subcore_barrier()
            pltpu.sync_copy(
                out_sh.at[pl.ds(rbase, ROWS_PT), :],
                sedge_hbm.at[q, pl.ds(rbase, ROWS_PT), :])
            plsc.subcore_barrier()

        for cv in range(NCORE):
            @pl.when(core == cv)
            def _():
                for p in range(2):
                    run_pass(2 * cv + p)

    return k(row, col, ewf, xws4, zeros128)


def kernel(x, edge_index, edge_attr, W_conv, b_conv, W_edge, b_edge, W_node, b_node):
    row = edge_index[0]
    col = edge_index[1]
    ew16 = _edge_props(edge_attr, W_edge, b_edge)
    xw = _xw(x, W_conv)
    zeros128 = jnp.zeros((ROWS_PT, 2 * D_LAT_N), jnp.float32)
    ewf = ew16.reshape(E * 2 * D_LAT_E)
    degpart = _deg_sc(col, ewf, zeros128)
    dis16, xws4 = _post_deg(degpart, xw)
    sedge = _msg_sc(row, col, ewf, xws4, zeros128)
    return _final(sedge, dis16, xw, W_node, b_conv, b_node)


# speedup vs baseline: 39.6145x; 1.0012x over previous
import functools

import jax
import jax.numpy as jnp
from jax import lax
from jax.experimental import pallas as pl
from jax.experimental.pallas import tpu as pltpu
from jax.experimental.pallas import tpu_sc as plsc

N = 10000
E = 320000
D_IN = 128
D_LAT_N = 64
D_LAT_E = 8
D_EDGE = 16
D_OUT = 128

NTILE = 16
NCORE = 2
NPAD = 10240
CHUNK = 80
BLK_D = 400
BLK_M = 80
ROWS_PT = NPAD // NTILE
EDGES_PT = E // NTILE
EDGES_PT_DEG = E // (NTILE * NCORE)


def _pre_body(ea_ref, x_ref, we_ref, be_ref, wc_ref, ew_ref, xw_ref):
    g = jax.nn.relu(
        jnp.dot(ea_ref[...], we_ref[...], preferred_element_type=jnp.float32)
        + be_ref[...]
    )
    ew_ref[...] = jnp.concatenate(
        [g, jnp.zeros((g.shape[0], D_LAT_E), jnp.float32)], axis=1)
    xw_ref[...] = jnp.dot(x_ref[...], wc_ref[...],
                          preferred_element_type=jnp.float32)


def _pre(edge_attr, x, W_edge, b_edge, W_conv):
    eblk = E // 25
    nblk = N // 25
    return pl.pallas_call(
        _pre_body,
        grid=(25,),
        in_specs=[
            pl.BlockSpec((eblk, D_EDGE), lambda i: (i, 0)),
            pl.BlockSpec((nblk, D_IN), lambda i: (i, 0)),
            pl.BlockSpec((D_EDGE, D_LAT_E), lambda i: (0, 0)),
            pl.BlockSpec((D_LAT_E,), lambda i: (0,)),
            pl.BlockSpec((D_IN, D_LAT_N), lambda i: (0, 0)),
        ],
        out_specs=[
            pl.BlockSpec((eblk, 2 * D_LAT_E), lambda i: (i, 0)),
            pl.BlockSpec((nblk, D_LAT_N), lambda i: (i, 0)),
        ],
        out_shape=[
            jax.ShapeDtypeStruct((E, 2 * D_LAT_E), jnp.float32),
            jax.ShapeDtypeStruct((N, D_LAT_N), jnp.float32),
        ],
    )(edge_attr, x, W_edge, b_edge, W_conv)


def _post_deg_body(dp_ref, xw_ref, dis_ref, xws_ref):
    deg = 1.0 + dp_ref[0][:, :D_LAT_E] + dp_ref[1][:, :D_LAT_E]
    dis = lax.rsqrt(deg)
    dis_ref[...] = jnp.concatenate([dis, dis], axis=1)
    xw = xw_ref[...]
    for q in range(4):
        for c in range(2):
            xws_ref[q, :, c * D_LAT_N:(c + 1) * D_LAT_N] = (
                dis[:, 2 * q + c:2 * q + c + 1] * xw)


def _post_deg(degpart, xw):
    blk = 400
    return pl.pallas_call(
        _post_deg_body,
        grid=(N // blk,),
        in_specs=[
            pl.BlockSpec((NCORE, blk, 2 * D_LAT_N), lambda i: (0, i, 0)),
            pl.BlockSpec((blk, D_LAT_N), lambda i: (i, 0)),
        ],
        out_specs=[
            pl.BlockSpec((blk, 2 * D_LAT_E), lambda i: (i, 0)),
            pl.BlockSpec((4, blk, 2 * D_LAT_N), lambda i: (0, i, 0)),
        ],
        out_shape=[
            jax.ShapeDtypeStruct((N, 2 * D_LAT_E), jnp.float32),
            jax.ShapeDtypeStruct((4, N, 2 * D_LAT_N), jnp.float32),
        ],
    )(degpart, xw)


def _final_body(sed_ref, dis_ref, xw_ref, wn_ref, bc_ref, bn_ref, o_ref):
    xw = xw_ref[...]
    bc = bc_ref[...][None, :]
    acc = jnp.broadcast_to(bn_ref[...][None, :], o_ref.shape).astype(jnp.float32)
    for q in range(4):
        sq = sed_ref[q]
        for cc in range(2):
            ch = 2 * q + cc
            d = dis_ref[:, ch][:, None]
            g = d * sq[:, cc * 64:(cc + 1) * 64] + (d * d) * xw + bc
            acc = acc + jnp.dot(
                g, wn_ref[ch * 64:(ch + 1) * 64, :],
                preferred_element_type=jnp.float32,
            )
    o_ref[...] = jax.nn.relu(acc)


def _final(sedge, dis, xw, W_node, b_conv, b_node):
    blk = 400
    return pl.pallas_call(
        _final_body,
        grid=(N // blk,),
        in_specs=[
            pl.BlockSpec((4, blk, 2 * D_LAT_N), lambda i: (0, i, 0)),
            pl.BlockSpec((blk, 2 * D_LAT_E), lambda i: (i, 0)),
            pl.BlockSpec((blk, D_LAT_N), lambda i: (i, 0)),
            pl.BlockSpec((D_LAT_E * D_LAT_N, D_OUT), lambda i: (0, 0)),
            pl.BlockSpec((D_LAT_N,), lambda i: (0,)),
            pl.BlockSpec((D_OUT,), lambda i: (0,)),
        ],
        out_specs=pl.BlockSpec((blk, D_OUT), lambda i: (i, 0)),
        out_shape=jax.ShapeDtypeStruct((N, D_OUT), jnp.float32),
    )(sedge, dis, xw, W_node, b_conv, b_node)


def _repack_idx(flat_ref, idx2_ref, nchunk):
    for s in range(nchunk):
        for j in range(CHUNK // 16):
            idx2_ref[s, pl.ds(j * 16, 16)] = flat_ref[pl.ds(s * CHUNK + j * 16, 16)]


def _deg_sc(col, ew16, zeros128):
    mesh = plsc.VectorSubcoreMesh(core_axis_name="c", subcore_axis_name="s")

    @functools.partial(
        pl.kernel,
        out_type=jax.ShapeDtypeStruct((NCORE, NPAD, 2 * D_LAT_N), jnp.float32),
        mesh=mesh,
        scratch_types=[
            pltpu.MemorySpace.VMEM_SHARED((NPAD, 2 * D_LAT_N), jnp.float32),
            pltpu.VMEM((BLK_D,), jnp.int32),
            pltpu.VMEM((BLK_D // CHUNK, CHUNK), jnp.int32),
            pltpu.VMEM((BLK_D * 2 * D_LAT_E,), jnp.float32),
            pltpu.VMEM((CHUNK, 2 * D_LAT_N), jnp.float32),
            pltpu.SemaphoreType.DMA,
            pltpu.SemaphoreType.DMA,
        ],
    )
    def k(col_hbm, ew_hbm, zeros_hbm, degpart_hbm, deg_sh, colb1, colb, ewb,
          ewpad, lsem, ssem):
        core = lax.axis_index("c")
        tile = lax.axis_index("s")
        rbase = tile * ROWS_PT
        pltpu.sync_copy(zeros_hbm, deg_sh.at[pl.ds(rbase, ROWS_PT), :])

        def zpad(i, _):
            for j in range(8):
                ewpad[i, pl.ds(j * 16, 16)] = jnp.zeros((16,), jnp.float32)
            return 0
        lax.fori_loop(0, CHUNK, zpad, 0)
        plsc.subcore_barrier()

        nblocks = EDGES_PT_DEG // BLK_D
        ebase0 = core * (E // NCORE) + tile * EDGES_PT_DEG

        def body(b, _):
            d1 = pltpu.async_copy(
                col_hbm.at[pl.ds(ebase0 + b * BLK_D, BLK_D)], colb1, lsem)
            d2 = pltpu.async_copy(
                ew_hbm.at[pl.ds((ebase0 + b * BLK_D) * 16, BLK_D * 16)],
                ewb, lsem)
            d1.wait()
            d2.wait()
            _repack_idx(colb1, colb, BLK_D // CHUNK)

            for s in range(BLK_D // CHUNK):
                def pbody(i, _, s=s):
                    ewpad[i, pl.ds(0, 16)] = ewb[pl.ds((s * CHUNK + i) * 16, 16)]
                    return 0
                lax.fori_loop(0, CHUNK, pbody, 0)
                pltpu.async_copy(
                    ewpad, deg_sh.at[colb.at[s]], ssem, add=True).wait()
            return 0

        lax.fori_loop(0, nblocks, body, 0)
        plsc.subcore_barrier()
        pltpu.sync_copy(
            deg_sh.at[pl.ds(rbase, ROWS_PT), :],
            degpart_hbm.at[core, pl.ds(rbase, ROWS_PT), :])

    return k(col, ew16, zeros128)


def _msg_sc(row, col, ewf, xws4, zeros128):
    mesh = plsc.VectorSubcoreMesh(core_axis_name="c", subcore_axis_name="s")
    nblocks = EDGES_PT // BLK_M

    @functools.partial(
        pl.kernel,
        out_type=jax.ShapeDtypeStruct((4, NPAD, 2 * D_LAT_N), jnp.float32),
        mesh=mesh,
        scratch_types=[
            pltpu.MemorySpace.VMEM_SHARED((NPAD, 2 * D_LAT_N), jnp.float32),
            pltpu.VMEM((2, 1, BLK_M), jnp.int32),
            pltpu.VMEM((4, 1, BLK_M), jnp.int32),
            pltpu.VMEM((BLK_M * 2 * D_LAT_E,), jnp.float32),
            pltpu.VMEM((BLK_M * 2 * D_LAT_E,), jnp.float32),
            pltpu.VMEM((BLK_M, 2 * D_LAT_N), jnp.float32),
            pltpu.VMEM((BLK_M, 2 * D_LAT_N), jnp.float32),
            pltpu.VMEM((BLK_M, 2 * D_LAT_N), jnp.float32),
            pltpu.VMEM((BLK_M, 2 * D_LAT_N), jnp.float32),
            pltpu.SemaphoreType.DMA,
            pltpu.SemaphoreType.DMA,
            pltpu.SemaphoreType.DMA,
            pltpu.SemaphoreType.DMA,
            pltpu.SemaphoreType.DMA,
            pltpu.SemaphoreType.DMA,
        ],
    )
    def k(row_hbm, col_hbm, ew_hbm, xws_hbm, zeros_hbm, sedge_hbm,
          out_sh, rowb, colb, ewb0, ewb1, xwsb0, xwsb1, msgb0, msgb1,
          lsem0, lsem1, gsem0, gsem1, ssem0, ssem1):
        core = lax.axis_index("c")
        tile = lax.axis_index("s")
        rbase = tile * ROWS_PT
        ebase0 = tile * EDGES_PT
        ewb = (ewb0, ewb1)
        xwsb = (xwsb0, xwsb1)
        msgb = (msgb0, msgb1)
        lsem = (lsem0, lsem1)
        gsem = (gsem0, gsem1)
        ssem = (ssem0, ssem1)
        zrows = zeros_hbm.at[pl.ds(0, BLK_M), :]

        def issue_l(b, k_):
            est = ebase0 + b * BLK_M
            pltpu.async_copy(row_hbm.at[pl.ds(est, BLK_M)],
                             rowb.at[k_, 0], lsem[k_])
            pltpu.async_copy(col_hbm.at[pl.ds(est, BLK_M)],
                             colb.at[lax.rem(b, 4), 0], lsem[k_])
            pltpu.async_copy(
                ew_hbm.at[pl.ds(est * 2 * D_LAT_E, BLK_M * 2 * D_LAT_E)],
                ewb[k_], lsem[k_])

        def drain_l(k_):
            pltpu.make_async_copy(row_hbm.at[pl.ds(0, BLK_M)],
                                  rowb.at[k_, 0], lsem[k_]).wait()
            pltpu.make_async_copy(col_hbm.at[pl.ds(0, BLK_M)],
                                  colb.at[0, 0], lsem[k_]).wait()
            pltpu.make_async_copy(
                ew_hbm.at[pl.ds(0, BLK_M * 2 * D_LAT_E)],
                ewb[k_], lsem[k_]).wait()

        def run_pass(q):
            def issue_g(k_):
                pltpu.async_copy(xws_hbm.at[q].at[rowb.at[k_, 0]],
                                 xwsb[k_], gsem[k_])

            def drain_g(k_):
                pltpu.make_async_copy(zrows, xwsb[k_], gsem[k_]).wait()

            def issue_s(b, k_):
                pltpu.async_copy(msgb[k_],
                                 out_sh.at[colb.at[lax.rem(b, 4), 0]],
                                 ssem[k_], add=True)

            def drain_s(k_):
                pltpu.make_async_copy(zrows, msgb[k_], ssem[k_]).wait()

            def compute(k_):
                def mbody(g, _):
                    for i in range(16):
                        e = g * 16 + i
                        wv = ewb[k_][pl.ds(e * 2 * D_LAT_E, 16)]
                        a0 = wv[2 * q]
                        a1 = wv[2 * q + 1]
                        for r in range(4):
                            msgb[k_][e, pl.ds(r * 16, 16)] = (
                                a0 * xwsb[k_][e, pl.ds(r * 16, 16)])
                            msgb[k_][e, pl.ds(64 + r * 16, 16)] = (
                                a1 * xwsb[k_][e, pl.ds(64 + r * 16, 16)])
                    return 0
                lax.fori_loop(0, BLK_M // 16, mbody, 0)

            pltpu.sync_copy(zeros_hbm, out_sh.at[pl.ds(rbase, ROWS_PT), :])
            plsc.subcore_barrier()

            issue_l(0, 0)
            drain_l(0)
            issue_g(0)
            issue_l(1, 1)

            def body(bb, _):
                for k_ in range(2):
                    b = 2 * bb + k_
                    kn = 1 - k_
                    drain_g(k_)

                    @pl.when(b + 1 < nblocks)
                    def _():
                        drain_l(kn)
                        issue_g(kn)

                    @pl.when(b >= 2)
                    def _():
                        drain_s(k_)

                    compute(k_)
                    issue_s(b, k_)

                    @pl.when(b + 2 < nblocks)
                    def _():
                        issue_l(b + 2, k_)
                return 0

            lax.fori_loop(0, nblocks // 2, body, 0)
            drain_s(0)
            drain_s(1)
            plsc.subcore_barrier()
            pltpu.sync_copy(
                out_sh.at[pl.ds(rbase, ROWS_PT), :],
                sedge_hbm.at[q, pl.ds(rbase, ROWS_PT), :])
            plsc.subcore_barrier()

        for cv in range(NCORE):
            @pl.when(core == cv)
            def _():
                for p in range(2):
                    run_pass(2 * cv + p)

    return k(row, col, ewf, xws4, zeros128)


def kernel(x, edge_index, edge_attr, W_conv, b_conv, W_edge, b_edge, W_node, b_node):
    row = edge_index[0]
    col = edge_index[1]
    ew16, xw = _pre(edge_attr, x, W_edge, b_edge, W_conv)
    zeros128 = jnp.zeros((ROWS_PT, 2 * D_LAT_N), jnp.float32)
    ewf = ew16.reshape(E * 2 * D_LAT_E)
    degpart = _deg_sc(col, ewf, zeros128)
    dis16, xws4 = _post_deg(degpart, xw)
    sedge = _msg_sc(row, col, ewf, xws4, zeros128)
    return _final(sedge, dis16, xw, W_node, b_conv, b_node)


# speedup vs baseline: 44.6593x; 1.1273x over previous
import functools

import jax
import jax.numpy as jnp
from jax import lax
from jax.experimental import pallas as pl
from jax.experimental.pallas import tpu as pltpu
from jax.experimental.pallas import tpu_sc as plsc

N = 10000
E = 320000
D_IN = 128
D_LAT_N = 64
D_LAT_E = 8
D_EDGE = 16
D_OUT = 128

NTILE = 16
NCORE = 2
NPAD = 10240
CHUNK = 80
BLK_D = 400
BLK_M = 80
ROWS_PT = NPAD // NTILE
EDGES_PT = E // NTILE
EDGES_PT_DEG = E // (NTILE * NCORE)


def _pre_body(ea2_ref, x_ref, w2_ref, b2_ref, wc_ref, ew2_ref, xw_ref):
    ew2_ref[...] = jax.nn.relu(
        jnp.dot(ea2_ref[...], w2_ref[...], preferred_element_type=jnp.float32)
        + b2_ref[...]
    )
    xw_ref[...] = jnp.dot(x_ref[...], wc_ref[...],
                          preferred_element_type=jnp.float32)


def _pre(edge_attr2, x, W2, b2, W_conv):
    eblk = E // 8 // 25
    nblk = N // 25
    return pl.pallas_call(
        _pre_body,
        grid=(25,),
        in_specs=[
            pl.BlockSpec((eblk, 8 * D_EDGE), lambda i: (i, 0)),
            pl.BlockSpec((nblk, D_IN), lambda i: (i, 0)),
            pl.BlockSpec((8 * D_EDGE, 8 * 2 * D_LAT_E), lambda i: (0, 0)),
            pl.BlockSpec((8 * 2 * D_LAT_E,), lambda i: (0,)),
            pl.BlockSpec((D_IN, D_LAT_N), lambda i: (0, 0)),
        ],
        out_specs=[
            pl.BlockSpec((eblk, 8 * 2 * D_LAT_E), lambda i: (i, 0)),
            pl.BlockSpec((nblk, D_LAT_N), lambda i: (i, 0)),
        ],
        out_shape=[
            jax.ShapeDtypeStruct((E // 8, 8 * 2 * D_LAT_E), jnp.float32),
            jax.ShapeDtypeStruct((N, D_LAT_N), jnp.float32),
        ],
    )(edge_attr2, x, W2, b2, W_conv)


def _post_deg_body(dp_ref, xw_ref, dis_ref, xws_ref):
    deg = 1.0 + dp_ref[0][:, :D_LAT_E] + dp_ref[1][:, :D_LAT_E]
    dis = lax.rsqrt(deg)
    dis_ref[...] = jnp.concatenate([dis, dis], axis=1)
    xw = xw_ref[...]
    for q in range(4):
        for c in range(2):
            xws_ref[q, :, c * D_LAT_N:(c + 1) * D_LAT_N] = (
                dis[:, 2 * q + c:2 * q + c + 1] * xw)


def _post_deg(degpart, xw):
    blk = 400
    return pl.pallas_call(
        _post_deg_body,
        grid=(N // blk,),
        in_specs=[
            pl.BlockSpec((NCORE, blk, 2 * D_LAT_N), lambda i: (0, i, 0)),
            pl.BlockSpec((blk, D_LAT_N), lambda i: (i, 0)),
        ],
        out_specs=[
            pl.BlockSpec((blk, 2 * D_LAT_E), lambda i: (i, 0)),
            pl.BlockSpec((4, blk, 2 * D_LAT_N), lambda i: (0, i, 0)),
        ],
        out_shape=[
            jax.ShapeDtypeStruct((N, 2 * D_LAT_E), jnp.float32),
            jax.ShapeDtypeStruct((4, N, 2 * D_LAT_N), jnp.float32),
        ],
    )(degpart, xw)


def _final_body(sed_ref, dis_ref, xw_ref, wn_ref, bc_ref, bn_ref, o_ref):
    xw = xw_ref[...]
    bc = bc_ref[...][None, :]
    acc = jnp.broadcast_to(bn_ref[...][None, :], o_ref.shape).astype(jnp.float32)
    for q in range(4):
        sq = sed_ref[q]
        for cc in range(2):
            ch = 2 * q + cc
            d = dis_ref[:, ch][:, None]
            g = d * sq[:, cc * 64:(cc + 1) * 64] + (d * d) * xw + bc
            acc = acc + jnp.dot(
                g, wn_ref[ch * 64:(ch + 1) * 64, :],
                preferred_element_type=jnp.float32,
            )
    o_ref[...] = jax.nn.relu(acc)


def _final(sedge, dis, xw, W_node, b_conv, b_node):
    blk = 400
    return pl.pallas_call(
        _final_body,
        grid=(N // blk,),
        in_specs=[
            pl.BlockSpec((4, blk, 2 * D_LAT_N), lambda i: (0, i, 0)),
            pl.BlockSpec((blk, 2 * D_LAT_E), lambda i: (i, 0)),
            pl.BlockSpec((blk, D_LAT_N), lambda i: (i, 0)),
            pl.BlockSpec((D_LAT_E * D_LAT_N, D_OUT), lambda i: (0, 0)),
            pl.BlockSpec((D_LAT_N,), lambda i: (0,)),
            pl.BlockSpec((D_OUT,), lambda i: (0,)),
        ],
        out_specs=pl.BlockSpec((blk, D_OUT), lambda i: (i, 0)),
        out_shape=jax.ShapeDtypeStruct((N, D_OUT), jnp.float32),
    )(sedge, dis, xw, W_node, b_conv, b_node)


def _repack_idx(flat_ref, idx2_ref, nchunk):
    for s in range(nchunk):
        for j in range(CHUNK // 16):
            idx2_ref[s, pl.ds(j * 16, 16)] = flat_ref[pl.ds(s * CHUNK + j * 16, 16)]


def _deg_sc(col, ew16, zeros128):
    mesh = plsc.VectorSubcoreMesh(core_axis_name="c", subcore_axis_name="s")

    @functools.partial(
        pl.kernel,
        out_type=jax.ShapeDtypeStruct((NCORE, NPAD, 2 * D_LAT_N), jnp.float32),
        mesh=mesh,
        scratch_types=[
            pltpu.MemorySpace.VMEM_SHARED((NPAD, 2 * D_LAT_N), jnp.float32),
            pltpu.VMEM((BLK_D,), jnp.int32),
            pltpu.VMEM((BLK_D // CHUNK, CHUNK), jnp.int32),
            pltpu.VMEM((BLK_D * 2 * D_LAT_E,), jnp.float32),
            pltpu.VMEM((CHUNK, 2 * D_LAT_N), jnp.float32),
            pltpu.SemaphoreType.DMA,
            pltpu.SemaphoreType.DMA,
        ],
    )
    def k(col_hbm, ew_hbm, zeros_hbm, degpart_hbm, deg_sh, colb1, colb, ewb,
          ewpad, lsem, ssem):
        core = lax.axis_index("c")
        tile = lax.axis_index("s")
        rbase = tile * ROWS_PT
        pltpu.sync_copy(zeros_hbm, deg_sh.at[pl.ds(rbase, ROWS_PT), :])

        def zpad(i, _):
            for j in range(8):
                ewpad[i, pl.ds(j * 16, 16)] = jnp.zeros((16,), jnp.float32)
            return 0
        lax.fori_loop(0, CHUNK, zpad, 0)
        plsc.subcore_barrier()

        nblocks = EDGES_PT_DEG // BLK_D
        ebase0 = core * (E // NCORE) + tile * EDGES_PT_DEG

        def body(b, _):
            d1 = pltpu.async_copy(
                col_hbm.at[pl.ds(ebase0 + b * BLK_D, BLK_D)], colb1, lsem)
            d2 = pltpu.async_copy(
                ew_hbm.at[pl.ds((ebase0 + b * BLK_D) * 16, BLK_D * 16)],
                ewb, lsem)
            d1.wait()
            d2.wait()
            _repack_idx(colb1, colb, BLK_D // CHUNK)

            for s in range(BLK_D // CHUNK):
                def pbody(i, _, s=s):
                    ewpad[i, pl.ds(0, 16)] = ewb[pl.ds((s * CHUNK + i) * 16, 16)]
                    return 0
                lax.fori_loop(0, CHUNK, pbody, 0)
                pltpu.async_copy(
                    ewpad, deg_sh.at[colb.at[s]], ssem, add=True).wait()
            return 0

        lax.fori_loop(0, nblocks, body, 0)
        plsc.subcore_barrier()
        pltpu.sync_copy(
            deg_sh.at[pl.ds(rbase, ROWS_PT), :],
            degpart_hbm.at[core, pl.ds(rbase, ROWS_PT), :])

    return k(col, ew16, zeros128)


def _msg_sc(row, col, ewf, xws4, zeros128):
    mesh = plsc.VectorSubcoreMesh(core_axis_name="c", subcore_axis_name="s")
    nblocks = EDGES_PT // BLK_M

    @functools.partial(
        pl.kernel,
        out_type=jax.ShapeDtypeStruct((4, NPAD, 2 * D_LAT_N), jnp.float32),
        mesh=mesh,
        scratch_types=[
            pltpu.MemorySpace.VMEM_SHARED((NPAD, 2 * D_LAT_N), jnp.float32),
            pltpu.VMEM((2, 1, BLK_M), jnp.int32),
            pltpu.VMEM((4, 1, BLK_M), jnp.int32),
            pltpu.VMEM((BLK_M * 2 * D_LAT_E,), jnp.float32),
            pltpu.VMEM((BLK_M * 2 * D_LAT_E,), jnp.float32),
            pltpu.VMEM((BLK_M, 2 * D_LAT_N), jnp.float32),
            pltpu.VMEM((BLK_M, 2 * D_LAT_N), jnp.float32),
            pltpu.VMEM((BLK_M, 2 * D_LAT_N), jnp.float32),
            pltpu.VMEM((BLK_M, 2 * D_LAT_N), jnp.float32),
            pltpu.SemaphoreType.DMA,
            pltpu.SemaphoreType.DMA,
            pltpu.SemaphoreType.DMA,
            pltpu.SemaphoreType.DMA,
            pltpu.SemaphoreType.DMA,
            pltpu.SemaphoreType.DMA,
        ],
    )
    def k(row_hbm, col_hbm, ew_hbm, xws_hbm, zeros_hbm, sedge_hbm,
          out_sh, rowb, colb, ewb0, ewb1, xwsb0, xwsb1, msgb0, msgb1,
          lsem0, lsem1, gsem0, gsem1, ssem0, ssem1):
        core = lax.axis_index("c")
        tile = lax.axis_index("s")
        rbase = tile * ROWS_PT
        ebase0 = tile * EDGES_PT
        ewb = (ewb0, ewb1)
        xwsb = (xwsb0, xwsb1)
        msgb = (msgb0, msgb1)
        lsem = (lsem0, lsem1)
        gsem = (gsem0, gsem1)
        ssem = (ssem0, ssem1)
        zrows = zeros_hbm.at[pl.ds(0, BLK_M), :]

        def issue_l(b, k_):
            est = ebase0 + b * BLK_M
            pltpu.async_copy(row_hbm.at[pl.ds(est, BLK_M)],
                             rowb.at[k_, 0], lsem[k_])
            pltpu.async_copy(col_hbm.at[pl.ds(est, BLK_M)],
                             colb.at[lax.rem(b, 4), 0], lsem[k_])
            pltpu.async_copy(
                ew_hbm.at[pl.ds(est * 2 * D_LAT_E, BLK_M * 2 * D_LAT_E)],
                ewb[k_], lsem[k_])

        def drain_l(k_):
            pltpu.make_async_copy(row_hbm.at[pl.ds(0, BLK_M)],
                                  rowb.at[k_, 0], lsem[k_]).wait()
            pltpu.make_async_copy(col_hbm.at[pl.ds(0, BLK_M)],
                                  colb.at[0, 0], lsem[k_]).wait()
            pltpu.make_async_copy(
                ew_hbm.at[pl.ds(0, BLK_M * 2 * D_LAT_E)],
                ewb[k_], lsem[k_]).wait()

        def run_pass(q):
            def issue_g(k_):
                pltpu.async_copy(xws_hbm.at[q].at[rowb.at[k_, 0]],
                                 xwsb[k_], gsem[k_])

            def drain_g(k_):
                pltpu.make_async_copy(zrows, xwsb[k_], gsem[k_]).wait()

            def issue_s(b, k_):
                pltpu.async_copy(msgb[k_],
                                 out_sh.at[colb.at[lax.rem(b, 4), 0]],
                                 ssem[k_], add=True)

            def drain_s(k_):
                pltpu.make_async_copy(zrows, msgb[k_], ssem[k_]).wait()

            def compute(k_):
                def mbody(g, _):
                    for i in range(16):
                        e = g * 16 + i
                        wv = ewb[k_][pl.ds(e * 2 * D_LAT_E, 16)]
                        a0 = wv[2 * q]
                        a1 = wv[2 * q + 1]
                        for r in range(4):
                            msgb[k_][e, pl.ds(r * 16, 16)] = (
                                a0 * xwsb[k_][e, pl.ds(r * 16, 16)])
                            msgb[k_][e, pl.ds(64 + r * 16, 16)] = (
                                a1 * xwsb[k_][e, pl.ds(64 + r * 16, 16)])
                    return 0
                lax.fori_loop(0, BLK_M // 16, mbody, 0)

            pltpu.sync_copy(zeros_hbm, out_sh.at[pl.ds(rbase, ROWS_PT), :])
            plsc.subcore_barrier()

            issue_l(0, 0)
            drain_l(0)
            issue_g(0)
            issue_l(1, 1)

            def body(bb, _):
                for k_ in range(2):
                    b = 2 * bb + k_
                    kn = 1 - k_
                    drain_g(k_)

                    @pl.when(b + 1 < nblocks)
                    def _():
                        drain_l(kn)
                        issue_g(kn)

                    @pl.when(b >= 2)
                    def _():
                        drain_s(k_)

                    compute(k_)
                    issue_s(b, k_)

                    @pl.when(b + 2 < nblocks)
                    def _():
                        issue_l(b + 2, k_)
                return 0

            lax.fori_loop(0, nblocks // 2, body, 0)
            drain_s(0)
            drain_s(1)
            plsc.subcore_barrier()
            pltpu.sync_copy(
                out_sh.at[pl.ds(rbase, ROWS_PT), :],
                sedge_hbm.at[q, pl.ds(rbase, ROWS_PT), :])
            plsc.subcore_barrier()

        for cv in range(NCORE):
            @pl.when(core == cv)
            def _():
                for p in range(2):
                    run_pass(2 * cv + p)

    return k(row, col, ewf, xws4, zeros128)


def kernel(x, edge_index, edge_attr, W_conv, b_conv, W_edge, b_edge, W_node, b_node):
    row = edge_index[0]
    col = edge_index[1]
    wpad = jnp.concatenate(
        [W_edge, jnp.zeros((D_EDGE, D_LAT_E), jnp.float32)], axis=1)
    bpad = jnp.concatenate([b_edge, jnp.zeros((D_LAT_E,), jnp.float32)])
    w2 = jnp.kron(jnp.eye(8, dtype=jnp.float32), wpad)
    b2 = jnp.tile(bpad, 8)
    ew2, xw = _pre(edge_attr.reshape(E // 8, 8 * D_EDGE), x, w2, b2, W_conv)
    zeros128 = jnp.zeros((ROWS_PT, 2 * D_LAT_N), jnp.float32)
    ewf = ew2.reshape(E * 2 * D_LAT_E)
    degpart = _deg_sc(col, ewf, zeros128)
    dis16, xws4 = _post_deg(degpart, xw)
    sedge = _msg_sc(row, col, ewf, xws4, zeros128)
    return _final(sedge, dis16, xw, W_node, b_conv, b_node)


# speedup vs baseline: 47.0580x; 1.0537x over previous
import functools

import jax
import jax.numpy as jnp
from jax import lax
from jax.experimental import pallas as pl
from jax.experimental.pallas import tpu as pltpu
from jax.experimental.pallas import tpu_sc as plsc

N = 10000
E = 320000
D_IN = 128
D_LAT_N = 64
D_LAT_E = 8
D_EDGE = 16
D_OUT = 128

NTILE = 16
NCORE = 2
NPAD = 10240
CHUNK = 80
BLK_D = 400
BLK_M = 80
ROWS_PT = NPAD // NTILE
EDGES_PT = E // NTILE
EDGES_PT_DEG = E // (NTILE * NCORE)


def _pre_body(ea2_ref, x_ref, w2_ref, b2_ref, wc_ref, ew2_ref, xw_ref):
    ew2_ref[...] = jax.nn.relu(
        jnp.dot(ea2_ref[...], w2_ref[...], preferred_element_type=jnp.float32)
        + b2_ref[...]
    )
    xw_ref[...] = jnp.dot(x_ref[...], wc_ref[...],
                          preferred_element_type=jnp.float32)


def _pre(edge_attr2, x, W2, b2, W_conv):
    eblk = E // 8 // 25
    nblk = N // 25
    return pl.pallas_call(
        _pre_body,
        grid=(25,),
        in_specs=[
            pl.BlockSpec((eblk, 8 * D_EDGE), lambda i: (i, 0)),
            pl.BlockSpec((nblk, D_IN), lambda i: (i, 0)),
            pl.BlockSpec((8 * D_EDGE, 8 * 2 * D_LAT_E), lambda i: (0, 0)),
            pl.BlockSpec((8 * 2 * D_LAT_E,), lambda i: (0,)),
            pl.BlockSpec((D_IN, D_LAT_N), lambda i: (0, 0)),
        ],
        out_specs=[
            pl.BlockSpec((eblk, 8 * 2 * D_LAT_E), lambda i: (i, 0)),
            pl.BlockSpec((nblk, D_LAT_N), lambda i: (i, 0)),
        ],
        out_shape=[
            jax.ShapeDtypeStruct((E // 8, 8 * 2 * D_LAT_E), jnp.float32),
            jax.ShapeDtypeStruct((N, D_LAT_N), jnp.float32),
        ],
    )(edge_attr2, x, W2, b2, W_conv)


def _post_deg_body(dp_ref, xw_ref, dis_ref, xws_ref):
    deg = 1.0 + dp_ref[0][:, :D_LAT_E] + dp_ref[1][:, :D_LAT_E]
    dis = lax.rsqrt(deg)
    dis_ref[...] = jnp.concatenate([dis, dis], axis=1)
    xw = xw_ref[...]
    for q in range(4):
        for c in range(2):
            xws_ref[q, :, c * D_LAT_N:(c + 1) * D_LAT_N] = (
                dis[:, 2 * q + c:2 * q + c + 1] * xw)


def _post_deg(degpart, xw):
    blk = 400
    return pl.pallas_call(
        _post_deg_body,
        grid=(N // blk,),
        in_specs=[
            pl.BlockSpec((NCORE, blk, 2 * D_LAT_N), lambda i: (0, i, 0)),
            pl.BlockSpec((blk, D_LAT_N), lambda i: (i, 0)),
        ],
        out_specs=[
            pl.BlockSpec((blk, 2 * D_LAT_E), lambda i: (i, 0)),
            pl.BlockSpec((4, blk, 2 * D_LAT_N), lambda i: (0, i, 0)),
        ],
        out_shape=[
            jax.ShapeDtypeStruct((N, 2 * D_LAT_E), jnp.float32),
            jax.ShapeDtypeStruct((4, N, 2 * D_LAT_N), jnp.float32),
        ],
    )(degpart, xw)


def _final_body(sed_ref, dis_ref, xw_ref, wn_ref, bc_ref, bn_ref, o_ref):
    xw = xw_ref[...]
    bc = bc_ref[...][None, :]
    acc = jnp.broadcast_to(bn_ref[...][None, :], o_ref.shape).astype(jnp.float32)
    for q in range(4):
        sq = sed_ref[q]
        for cc in range(2):
            ch = 2 * q + cc
            d = dis_ref[:, ch][:, None]
            g = d * sq[:, cc * 64:(cc + 1) * 64] + (d * d) * xw + bc
            acc = acc + jnp.dot(
                g, wn_ref[ch * 64:(ch + 1) * 64, :],
                preferred_element_type=jnp.float32,
            )
    o_ref[...] = jax.nn.relu(acc)


def _final(sedge, dis, xw, W_node, b_conv, b_node):
    blk = 400
    return pl.pallas_call(
        _final_body,
        grid=(N // blk,),
        in_specs=[
            pl.BlockSpec((4, blk, 2 * D_LAT_N), lambda i: (0, i, 0)),
            pl.BlockSpec((blk, 2 * D_LAT_E), lambda i: (i, 0)),
            pl.BlockSpec((blk, D_LAT_N), lambda i: (i, 0)),
            pl.BlockSpec((D_LAT_E * D_LAT_N, D_OUT), lambda i: (0, 0)),
            pl.BlockSpec((D_LAT_N,), lambda i: (0,)),
            pl.BlockSpec((D_OUT,), lambda i: (0,)),
        ],
        out_specs=pl.BlockSpec((blk, D_OUT), lambda i: (i, 0)),
        out_shape=jax.ShapeDtypeStruct((N, D_OUT), jnp.float32),
    )(sedge, dis, xw, W_node, b_conv, b_node)


def _repack_idx(flat_ref, idx2_ref, nchunk):
    for s in range(nchunk):
        for j in range(CHUNK // 16):
            idx2_ref[s, pl.ds(j * 16, 16)] = flat_ref[pl.ds(s * CHUNK + j * 16, 16)]


def _deg_sc(col, ewf, zeros128):
    mesh = plsc.VectorSubcoreMesh(core_axis_name="c", subcore_axis_name="s")
    nchunks = EDGES_PT_DEG // CHUNK

    @functools.partial(
        pl.kernel,
        out_type=jax.ShapeDtypeStruct((NCORE, NPAD, 2 * D_LAT_N), jnp.float32),
        mesh=mesh,
        scratch_types=[
            pltpu.MemorySpace.VMEM_SHARED((NPAD, 2 * D_LAT_N), jnp.float32),
            pltpu.VMEM((4, 1, CHUNK), jnp.int32),
            pltpu.VMEM((CHUNK * 2 * D_LAT_E,), jnp.float32),
            pltpu.VMEM((CHUNK * 2 * D_LAT_E,), jnp.float32),
            pltpu.VMEM((CHUNK, 2 * D_LAT_N), jnp.float32),
            pltpu.VMEM((CHUNK, 2 * D_LAT_N), jnp.float32),
            pltpu.SemaphoreType.DMA,
            pltpu.SemaphoreType.DMA,
            pltpu.SemaphoreType.DMA,
            pltpu.SemaphoreType.DMA,
        ],
    )
    def k(col_hbm, ew_hbm, zeros_hbm, degpart_hbm, deg_sh, colb, ewb0, ewb1,
          ewpad0, ewpad1, lsem0, lsem1, ssem0, ssem1):
        core = lax.axis_index("c")
        tile = lax.axis_index("s")
        rbase = tile * ROWS_PT
        ebase0 = core * (E // NCORE) + tile * EDGES_PT_DEG
        ewb = (ewb0, ewb1)
        ewpad = (ewpad0, ewpad1)
        lsem = (lsem0, lsem1)
        ssem = (ssem0, ssem1)

        pltpu.sync_copy(zeros_hbm, deg_sh.at[pl.ds(rbase, ROWS_PT), :])
        for k_ in range(2):
            def zpad(i, _, k_=k_):
                for j in range(8):
                    ewpad[k_][i, pl.ds(j * 16, 16)] = jnp.zeros((16,),
                                                                jnp.float32)
                return 0
            lax.fori_loop(0, CHUNK, zpad, 0)
        plsc.subcore_barrier()

        def issue_l(c, k_):
            est = ebase0 + c * CHUNK
            pltpu.async_copy(col_hbm.at[pl.ds(est, CHUNK)],
                             colb.at[lax.rem(c, 4), 0], lsem[k_])
            pltpu.async_copy(
                ew_hbm.at[pl.ds(est * 2 * D_LAT_E, CHUNK * 2 * D_LAT_E)],
                ewb[k_], lsem[k_])

        def drain_l(k_):
            pltpu.make_async_copy(col_hbm.at[pl.ds(0, CHUNK)],
                                  colb.at[0, 0], lsem[k_]).wait()
            pltpu.make_async_copy(
                ew_hbm.at[pl.ds(0, CHUNK * 2 * D_LAT_E)],
                ewb[k_], lsem[k_]).wait()

        def pad(k_):
            def pbody(i, _):
                ewpad[k_][i, pl.ds(0, 16)] = ewb[k_][pl.ds(i * 2 * D_LAT_E, 16)]
                return 0
            lax.fori_loop(0, CHUNK, pbody, 0)

        def issue_s(c, k_):
            pltpu.async_copy(ewpad[k_],
                             deg_sh.at[colb.at[lax.rem(c, 4), 0]],
                             ssem[k_], add=True)

        def drain_s(k_):
            pltpu.make_async_copy(zeros_hbm.at[pl.ds(0, CHUNK), :],
                                  ewpad[k_], ssem[k_]).wait()

        issue_l(0, 0)
        issue_l(1, 1)

        def body(bb, _):
            for k_ in range(2):
                c = 2 * bb + k_
                drain_l(k_)

                @pl.when(c >= 2)
                def _():
                    drain_s(k_)

                pad(k_)
                issue_s(c, k_)

                @pl.when(c + 2 < nchunks)
                def _():
                    issue_l(c + 2, k_)
            return 0

        lax.fori_loop(0, nchunks // 2, body, 0)
        drain_l(0)
        drain_s(0)
        pad(0)
        issue_s(nchunks - 1, 0)
        drain_s(0)
        drain_s(1)
        plsc.subcore_barrier()
        pltpu.sync_copy(
            deg_sh.at[pl.ds(rbase, ROWS_PT), :],
            degpart_hbm.at[core, pl.ds(rbase, ROWS_PT), :])

    return k(col, ewf, zeros128)


def _msg_sc(row, col, ewf, xws4, zeros128):
    mesh = plsc.VectorSubcoreMesh(core_axis_name="c", subcore_axis_name="s")
    nblocks = EDGES_PT // BLK_M

    @functools.partial(
        pl.kernel,
        out_type=jax.ShapeDtypeStruct((4, NPAD, 2 * D_LAT_N), jnp.float32),
        mesh=mesh,
        scratch_types=[
            pltpu.MemorySpace.VMEM_SHARED((NPAD, 2 * D_LAT_N), jnp.float32),
            pltpu.VMEM((2, 1, BLK_M), jnp.int32),
            pltpu.VMEM((4, 1, BLK_M), jnp.int32),
            pltpu.VMEM((BLK_M * 2 * D_LAT_E,), jnp.float32),
            pltpu.VMEM((BLK_M * 2 * D_LAT_E,), jnp.float32),
            pltpu.VMEM((BLK_M, 2 * D_LAT_N), jnp.float32),
            pltpu.VMEM((BLK_M, 2 * D_LAT_N), jnp.float32),
            pltpu.VMEM((BLK_M, 2 * D_LAT_N), jnp.float32),
            pltpu.VMEM((BLK_M, 2 * D_LAT_N), jnp.float32),
            pltpu.SemaphoreType.DMA,
            pltpu.SemaphoreType.DMA,
            pltpu.SemaphoreType.DMA,
            pltpu.SemaphoreType.DMA,
            pltpu.SemaphoreType.DMA,
            pltpu.SemaphoreType.DMA,
        ],
    )
    def k(row_hbm, col_hbm, ew_hbm, xws_hbm, zeros_hbm, sedge_hbm,
          out_sh, rowb, colb, ewb0, ewb1, xwsb0, xwsb1, msgb0, msgb1,
          lsem0, lsem1, gsem0, gsem1, ssem0, ssem1):
        core = lax.axis_index("c")
        tile = lax.axis_index("s")
        rbase = tile * ROWS_PT
        ebase0 = tile * EDGES_PT
        ewb = (ewb0, ewb1)
        xwsb = (xwsb0, xwsb1)
        msgb = (msgb0, msgb1)
        lsem = (lsem0, lsem1)
        gsem = (gsem0, gsem1)
        ssem = (ssem0, ssem1)
        zrows = zeros_hbm.at[pl.ds(0, BLK_M), :]

        def issue_l(b, k_):
            est = ebase0 + b * BLK_M
            pltpu.async_copy(row_hbm.at[pl.ds(est, BLK_M)],
                             rowb.at[k_, 0], lsem[k_])
            pltpu.async_copy(col_hbm.at[pl.ds(est, BLK_M)],
                             colb.at[lax.rem(b, 4), 0], lsem[k_])
            pltpu.async_copy(
                ew_hbm.at[pl.ds(est * 2 * D_LAT_E, BLK_M * 2 * D_LAT_E)],
                ewb[k_], lsem[k_])

        def drain_l(k_):
            pltpu.make_async_copy(row_hbm.at[pl.ds(0, BLK_M)],
                                  rowb.at[k_, 0], lsem[k_]).wait()
            pltpu.make_async_copy(col_hbm.at[pl.ds(0, BLK_M)],
                                  colb.at[0, 0], lsem[k_]).wait()
            pltpu.make_async_copy(
                ew_hbm.at[pl.ds(0, BLK_M * 2 * D_LAT_E)],
                ewb[k_], lsem[k_]).wait()

        def run_pass(q):
            def issue_g(k_):
                pltpu.async_copy(xws_hbm.at[q].at[rowb.at[k_, 0]],
                                 xwsb[k_], gsem[k_])

            def drain_g(k_):
                pltpu.make_async_copy(zrows, xwsb[k_], gsem[k_]).wait()

            def issue_s(b, k_):
                pltpu.async_copy(msgb[k_],
                                 out_sh.at[colb.at[lax.rem(b, 4), 0]],
                                 ssem[k_], add=True)

            def drain_s(k_):
                pltpu.make_async_copy(zrows, msgb[k_], ssem[k_]).wait()

            def compute(k_):
                def mbody(g, _):
                    for i in range(16):
                        e = g * 16 + i
                        wv = ewb[k_][pl.ds(e * 2 * D_LAT_E, 16)]
                        a0 = wv[2 * q]
                        a1 = wv[2 * q + 1]
                        for r in range(4):
                            msgb[k_][e, pl.ds(r * 16, 16)] = (
                                a0 * xwsb[k_][e, pl.ds(r * 16, 16)])
                            msgb[k_][e, pl.ds(64 + r * 16, 16)] = (
                                a1 * xwsb[k_][e, pl.ds(64 + r * 16, 16)])
                    return 0
                lax.fori_loop(0, BLK_M // 16, mbody, 0)

            pltpu.sync_copy(zeros_hbm, out_sh.at[pl.ds(rbase, ROWS_PT), :])
            plsc.subcore_barrier()

            issue_l(0, 0)
            drain_l(0)
            issue_g(0)
            issue_l(1, 1)

            def body(bb, _):
                for k_ in range(2):
                    b = 2 * bb + k_
                    kn = 1 - k_
                    drain_g(k_)

                    @pl.when(b + 1 < nblocks)
                    def _():
                        drain_l(kn)
                        issue_g(kn)

                    @pl.when(b >= 2)
                    def _():
                        drain_s(k_)

                    compute(k_)
                    issue_s(b, k_)

                    @pl.when(b + 2 < nblocks)
                    def _():
                        issue_l(b + 2, k_)
                return 0

            lax.fori_loop(0, nblocks // 2, body, 0)
            drain_s(0)
            drain_s(1)
            plsc.subcore_barrier()
            pltpu.sync_copy(
                out_sh.at[pl.ds(rbase, ROWS_PT), :],
                sedge_hbm.at[q, pl.ds(rbase, ROWS_PT), :])
            plsc.subcore_barrier()

        for cv in range(NCORE):
            @pl.when(core == cv)
            def _():
                for p in range(2):
                    run_pass(2 * cv + p)

    return k(row, col, ewf, xws4, zeros128)


def kernel(x, edge_index, edge_attr, W_conv, b_conv, W_edge, b_edge, W_node, b_node):
    row = edge_index[0]
    col = edge_index[1]
    wpad = jnp.concatenate(
        [W_edge, jnp.zeros((D_EDGE, D_LAT_E), jnp.float32)], axis=1)
    bpad = jnp.concatenate([b_edge, jnp.zeros((D_LAT_E,), jnp.float32)])
    w2 = jnp.kron(jnp.eye(8, dtype=jnp.float32), wpad)
    b2 = jnp.tile(bpad, 8)
    ew2, xw = _pre(edge_attr.reshape(E // 8, 8 * D_EDGE), x, w2, b2, W_conv)
    zeros128 = jnp.zeros((ROWS_PT, 2 * D_LAT_N), jnp.float32)
    ewf = ew2.reshape(E * 2 * D_LAT_E)
    degpart = _deg_sc(col, ewf, zeros128)
    dis16, xws4 = _post_deg(degpart, xw)
    sedge = _msg_sc(row, col, ewf, xws4, zeros128)
    return _final(sedge, dis16, xw, W_node, b_conv, b_node)
